# baseline (speedup 1.0000x reference)
import dataclasses
import functools

import jax
import jax.numpy as jnp
from jax import lax
from jax.experimental import pallas as pl
from jax.experimental.pallas import tpu as pltpu
from jax.experimental.pallas import tpu_sc as plsc

N_CH = 1048576
NUM_PED = 16
PED_F = 4
SPA_F = 2
OUT_F = PED_F + SPA_F
OUT_R = 8

NC, NS, L = 2, 16, 16
NW = NC * NS
N_CHUNK = N_CH // 128
CHUNK_PER_W = N_CHUNK // NW
WG = 32
N_BLK = CHUNK_PER_W // WG


def _body(table_hbm, spatial_hbm, ped_hbm, out_hbm, table_v, rep_v,
          idx_v0, idx_v1, out_v0, out_v1, out_v2,
          sp0, sp1, ss0, ss1, ss2, so0, so1, so2):
    wid = lax.axis_index("s") * NC + lax.axis_index("c")
    w_base = wid * CHUNK_PER_W

    lanes = lax.iota(jnp.int32, L)

    pltpu.sync_copy(table_hbm, table_v)
    for q in range(NUM_PED * PED_F // L):
        v = table_v[pl.ds(q * L, L)]
        base = (lanes + q * L) * L
        for i in range(L):
            plsc.store_scatter(rep_v, [base + i], v)

    idx_bufs = (idx_v0, idx_v1)
    out_bufs = (out_v0, out_v1, out_v2)
    ped_sems = (sp0, sp1)
    spa_sems = (ss0, ss1, ss2)
    out_sems = (so0, so1, so2)

    def issue_in(blk):
        g0 = w_base + blk * WG
        ph = pltpu.async_copy(ped_hbm.at[pl.ds(g0 * 128, WG * 128)],
                              idx_bufs[blk % 2], ped_sems[blk % 2])
        sh = pltpu.async_copy(spatial_hbm.at[pl.ds(g0, WG)],
                              out_bufs[blk % 3].at[:, PED_F:PED_F + SPA_F, :],
                              spa_sems[blk % 3])
        return ph, sh

    in_dma = [None] * N_BLK
    out_dma = [None] * N_BLK
    in_dma[0] = issue_in(0)
    in_dma[1] = issue_in(1)

    for blk in range(N_BLK):
        idx_v = idx_bufs[blk % 2]
        out_v = out_bufs[blk % 3]
        g0 = w_base + blk * WG
        for h in in_dma[blk]:
            h.wait()

        @plsc.parallel_loop(0, WG, unroll=4)
        def _(c):
            for s in range(128 // L):
                p = idx_v[pl.ds(c * 128 + s * L, L)]
                a = p * (L * PED_F) + lanes
                for f in range(PED_F):
                    vals = plsc.load_gather(rep_v, [a + f * L])
                    out_v.at[c, f][pl.ds(s * L, L)] = vals

        out_dma[blk] = pltpu.async_copy(out_v, out_hbm.at[pl.ds(g0, WG)],
                                        out_sems[blk % 3])
        if blk + 2 < N_BLK:
            if blk >= 1:
                out_dma[blk - 1].wait()
            in_dma[blk + 2] = issue_in(blk + 2)

    for blk in range(max(N_BLK - 3, 0), N_BLK):
        out_dma[blk].wait()


def kernel(pedestal_table, spatial_embeddings, pedestals):
    mesh = plsc.VectorSubcoreMesh(core_axis_name="c", subcore_axis_name="s")
    cp = pltpu.CompilerParams()
    if "needs_layout_passes" in pltpu.CompilerParams.__dataclass_fields__:
        cp = dataclasses.replace(cp, needs_layout_passes=False)
    k = functools.partial(
        pl.kernel,
        out_type=jax.ShapeDtypeStruct((N_CHUNK, OUT_R, 128), jnp.float32),
        mesh=mesh,
        scratch_types=[
            pltpu.VMEM((NUM_PED * PED_F,), jnp.float32),
            pltpu.VMEM((NUM_PED * PED_F * L,), jnp.float32),
            pltpu.VMEM((WG * 128,), jnp.int32),
            pltpu.VMEM((WG * 128,), jnp.int32),
            pltpu.VMEM((WG, OUT_R, 128), jnp.float32),
            pltpu.VMEM((WG, OUT_R, 128), jnp.float32),
            pltpu.VMEM((WG, OUT_R, 128), jnp.float32),
        ] + [pltpu.SemaphoreType.DMA] * 8,
        compiler_params=cp,
    )(_body)
    spatial3 = spatial_embeddings.reshape(N_CHUNK, 128, SPA_F).transpose(0, 2, 1)
    out3 = k(pedestal_table.reshape(NUM_PED * PED_F), spatial3, pedestals)
    return out3.transpose(0, 2, 1).reshape(N_CH, OUT_R)[:, :OUT_F]


# speedup vs baseline: 1.0044x; 1.0044x over previous
import dataclasses
import functools

import jax
import jax.numpy as jnp
from jax import lax
from jax.experimental import pallas as pl
from jax.experimental.pallas import tpu as pltpu
from jax.experimental.pallas import tpu_sc as plsc

N_CH = 1048576
NUM_PED = 16
PED_F = 4
SPA_F = 2
OUT_F = PED_F + SPA_F
OUT_R = 8

NC, NS, L = 2, 16, 16
NW = NC * NS
N_CHUNK = N_CH // 128
CHUNK_PER_W = N_CHUNK // NW
WG = 32
N_BLK = CHUNK_PER_W // WG


def _body(table_hbm, spatial_hbm, ped_hbm, out_hbm, table_v, rep_v,
          idx_v0, idx_v1, out_v0, out_v1, out_v2,
          sp0, sp1, ss0, ss1, ss2, so0, so1, so2):
    wid = lax.axis_index("s") * NC + lax.axis_index("c")
    w_base = wid * CHUNK_PER_W

    lanes = lax.iota(jnp.int32, L)

    pltpu.sync_copy(table_hbm, table_v)
    for q in range(NUM_PED * PED_F // L):
        v = table_v[pl.ds(q * L, L)]
        base = (lanes + q * L) * L
        for i in range(L):
            plsc.store_scatter(rep_v, [base + i], v)

    idx_bufs = (idx_v0, idx_v1)
    out_bufs = (out_v0, out_v1, out_v2)
    ped_sems = (sp0, sp1)
    spa_sems = (ss0, ss1, ss2)
    out_sems = (so0, so1, so2)

    def issue_in(blk):
        g0 = w_base + blk * WG
        ph = pltpu.async_copy(ped_hbm.at[pl.ds(g0 * 128, WG * 128)],
                              idx_bufs[blk % 2], ped_sems[blk % 2])
        sh = pltpu.async_copy(spatial_hbm.at[pl.ds(g0, WG)],
                              out_bufs[blk % 3].at[:, PED_F:PED_F + SPA_F, :],
                              spa_sems[blk % 3])
        return ph, sh

    in_dma = [None] * N_BLK
    out_dma = [None] * N_BLK
    in_dma[0] = issue_in(0)
    in_dma[1] = issue_in(1)

    for blk in range(N_BLK):
        idx_v = idx_bufs[blk % 2]
        out_v = out_bufs[blk % 3]
        g0 = w_base + blk * WG
        for h in in_dma[blk]:
            h.wait()

        @plsc.parallel_loop(0, WG, unroll=2)
        def _(c):
            for s in range(128 // L):
                p = idx_v[pl.ds(c * 128 + s * L, L)]
                a = p * (L * PED_F) + lanes
                for f in range(PED_F):
                    vals = plsc.load_gather(rep_v, [a + f * L])
                    out_v.at[c, f][pl.ds(s * L, L)] = vals

        out_dma[blk] = pltpu.async_copy(out_v, out_hbm.at[pl.ds(g0, WG)],
                                        out_sems[blk % 3])
        if blk + 2 < N_BLK:
            if blk >= 1:
                out_dma[blk - 1].wait()
            in_dma[blk + 2] = issue_in(blk + 2)

    for blk in range(max(N_BLK - 3, 0), N_BLK):
        out_dma[blk].wait()


def kernel(pedestal_table, spatial_embeddings, pedestals):
    mesh = plsc.VectorSubcoreMesh(core_axis_name="c", subcore_axis_name="s")
    cp = pltpu.CompilerParams()
    if "needs_layout_passes" in pltpu.CompilerParams.__dataclass_fields__:
        cp = dataclasses.replace(cp, needs_layout_passes=False)
    k = functools.partial(
        pl.kernel,
        out_type=jax.ShapeDtypeStruct((N_CHUNK, OUT_R, 128), jnp.float32),
        mesh=mesh,
        scratch_types=[
            pltpu.VMEM((NUM_PED * PED_F,), jnp.float32),
            pltpu.VMEM((NUM_PED * PED_F * L,), jnp.float32),
            pltpu.VMEM((WG * 128,), jnp.int32),
            pltpu.VMEM((WG * 128,), jnp.int32),
            pltpu.VMEM((WG, OUT_R, 128), jnp.float32),
            pltpu.VMEM((WG, OUT_R, 128), jnp.float32),
            pltpu.VMEM((WG, OUT_R, 128), jnp.float32),
        ] + [pltpu.SemaphoreType.DMA] * 8,
        compiler_params=cp,
    )(_body)
    spatial3 = spatial_embeddings.reshape(N_CHUNK, 128, SPA_F).transpose(0, 2, 1)
    out3 = k(pedestal_table.reshape(NUM_PED * PED_F), spatial3, pedestals)
    return out3.transpose(0, 2, 1).reshape(N_CH, OUT_R)[:, :OUT_F]


# speedup vs baseline: 1.0581x; 1.0535x over previous
import dataclasses
import functools

import jax
import jax.numpy as jnp
from jax import lax
from jax.experimental import pallas as pl
from jax.experimental.pallas import tpu as pltpu
from jax.experimental.pallas import tpu_sc as plsc

N_CH = 1048576
NUM_PED = 16
PED_F = 4
SPA_F = 2
OUT_F = PED_F + SPA_F
OUT_R = 8

NC, NS, L = 2, 16, 16
NW = NC * NS
N_CHUNK = N_CH // 128
CHUNK_PER_W = N_CHUNK // NW
WG = 32
N_BLK = CHUNK_PER_W // WG


def _body(table_hbm, spatial_hbm, ped_hbm, out_hbm, table_v, rep_v,
          idx_v0, idx_v1, out_v0, out_v1, out_v2,
          sp0, sp1, ss0, ss1, ss2, so0, so1, so2):
    wid = lax.axis_index("s") * NC + lax.axis_index("c")
    w_base = wid * CHUNK_PER_W

    lanes = lax.iota(jnp.int32, L)

    pltpu.sync_copy(table_hbm, table_v)
    for q in range(NUM_PED * PED_F // L):
        v = table_v[pl.ds(q * L, L)]
        base = (lanes + q * L) * L
        for i in range(L):
            plsc.store_scatter(rep_v, [base + i], v)

    idx_bufs = (idx_v0, idx_v1)
    out_bufs = (out_v0, out_v1, out_v2)
    ped_sems = (sp0, sp1)
    spa_sems = (ss0, ss1, ss2)
    out_sems = (so0, so1, so2)

    def issue_in(blk):
        g0 = w_base + blk * WG
        ph = pltpu.async_copy(ped_hbm.at[pl.ds(g0 * 128, WG * 128)],
                              idx_bufs[blk % 2], ped_sems[blk % 2])
        sh = pltpu.async_copy(spatial_hbm.at[pl.ds(g0, WG)],
                              out_bufs[blk % 3].at[:, PED_F:PED_F + SPA_F, :],
                              spa_sems[blk % 3])
        return ph, sh

    in_dma = [None] * N_BLK
    out_dma = [None] * N_BLK
    in_dma[0] = issue_in(0)
    in_dma[1] = issue_in(1)

    for blk in range(N_BLK):
        idx_v = idx_bufs[blk % 2]
        out_v = out_bufs[blk % 3]
        g0 = w_base + blk * WG
        for h in in_dma[blk]:
            h.wait()

        @plsc.parallel_loop(0, WG, unroll=2)
        def _(c):
            for s in range(128 // L):
                p = idx_v[pl.ds(c * 128 + s * L, L)]
                a = p * (L * PED_F) + lanes
                for f in range(PED_F):
                    vals = plsc.load_gather(rep_v, [a + f * L])
                    out_v.at[c, f][pl.ds(s * L, L)] = vals

        out_dma[blk] = pltpu.async_copy(
            out_v.at[:, 0:OUT_F, :],
            out_hbm.at[pl.ds(g0, WG), 0:OUT_F],
            out_sems[blk % 3])
        if blk + 2 < N_BLK:
            if blk >= 1:
                out_dma[blk - 1].wait()
            in_dma[blk + 2] = issue_in(blk + 2)

    for blk in range(max(N_BLK - 3, 0), N_BLK):
        out_dma[blk].wait()


def kernel(pedestal_table, spatial_embeddings, pedestals):
    mesh = plsc.VectorSubcoreMesh(core_axis_name="c", subcore_axis_name="s")
    cp = pltpu.CompilerParams()
    if "needs_layout_passes" in pltpu.CompilerParams.__dataclass_fields__:
        cp = dataclasses.replace(cp, needs_layout_passes=False)
    k = functools.partial(
        pl.kernel,
        out_type=jax.ShapeDtypeStruct((N_CHUNK, OUT_R, 128), jnp.float32),
        mesh=mesh,
        scratch_types=[
            pltpu.VMEM((NUM_PED * PED_F,), jnp.float32),
            pltpu.VMEM((NUM_PED * PED_F * L,), jnp.float32),
            pltpu.VMEM((WG * 128,), jnp.int32),
            pltpu.VMEM((WG * 128,), jnp.int32),
            pltpu.VMEM((WG, OUT_R, 128), jnp.float32),
            pltpu.VMEM((WG, OUT_R, 128), jnp.float32),
            pltpu.VMEM((WG, OUT_R, 128), jnp.float32),
        ] + [pltpu.SemaphoreType.DMA] * 8,
        compiler_params=cp,
    )(_body)
    spatial3 = spatial_embeddings.reshape(N_CHUNK, 128, SPA_F).transpose(0, 2, 1)
    out3 = k(pedestal_table.reshape(NUM_PED * PED_F), spatial3, pedestals)
    return out3.transpose(0, 2, 1).reshape(N_CH, OUT_R)[:, :OUT_F]
